# Initial kernel scaffold; baseline (speedup 1.0000x reference)
import jax
import jax.numpy as jnp
from jax.experimental import pallas as pl


def kernel(x, positions, W1, b1, W2, b2):
    raise NotImplementedError("write your pallas kernel here")


# speedup vs baseline: 456.6757x; 456.6757x over previous
import jax
import jax.numpy as jnp
from jax.experimental import pallas as pl
from jax.experimental.pallas import tpu as pltpu

_OBS_DIM = 512
_HIDDEN_DIM = 512
_NUM_ENVS = 64
_N_AGENTS = 128
_DIST = 0.2


def _fused_env_kernel(pos_ref, post_ref, x_ref, w1_ref, b1_ref, w2_ref,
                      b2_ref, out_ref):
    pos = pos_ref[0]
    post = post_ref[0]
    xe = x_ref[0]

    dx = pos[:, 0:1] - post[0:1, :]
    dy = pos[:, 1:2] - post[1:2, :]
    dist = jnp.sqrt(dx * dx + dy * dy)
    maskf = (dist <= _DIST).astype(jnp.float32)

    deg = jnp.maximum(jnp.sum(maskf, axis=1, keepdims=True), 1.0)

    agg = jnp.dot(maskf, xe, preferred_element_type=jnp.float32) / deg
    h1 = jnp.maximum(
        jnp.dot(agg, w1_ref[...], preferred_element_type=jnp.float32)
        + b1_ref[...], 0.0)

    agg2 = jnp.dot(maskf, h1, preferred_element_type=jnp.float32) / deg
    h2 = jnp.maximum(
        jnp.dot(agg2, w2_ref[...], preferred_element_type=jnp.float32)
        + b2_ref[...], 0.0)

    out_ref[...] = h2


def kernel(x, positions, W1, b1, W2, b2):
    num_envs, n_agents, feat = x.shape
    pos_t = positions.transpose(0, 2, 1)
    b1r = b1.reshape(1, _HIDDEN_DIM)
    b2r = b2.reshape(1, _HIDDEN_DIM)

    out = pl.pallas_call(
        _fused_env_kernel,
        grid=(num_envs,),
        in_specs=[
            pl.BlockSpec((1, n_agents, 2), lambda e: (e, 0, 0)),
            pl.BlockSpec((1, 2, n_agents), lambda e: (e, 0, 0)),
            pl.BlockSpec((1, n_agents, feat), lambda e: (e, 0, 0)),
            pl.BlockSpec((feat, _HIDDEN_DIM), lambda e: (0, 0)),
            pl.BlockSpec((1, _HIDDEN_DIM), lambda e: (0, 0)),
            pl.BlockSpec((_HIDDEN_DIM, _HIDDEN_DIM), lambda e: (0, 0)),
            pl.BlockSpec((1, _HIDDEN_DIM), lambda e: (0, 0)),
        ],
        out_specs=pl.BlockSpec((n_agents, _HIDDEN_DIM), lambda e: (e, 0)),
        out_shape=jax.ShapeDtypeStruct((num_envs * n_agents, _HIDDEN_DIM),
                                       jnp.float32),
        compiler_params=pltpu.CompilerParams(
            dimension_semantics=("parallel",)),
    )(positions, pos_t, x, W1, b1r, W2, b2r)
    return out


# speedup vs baseline: 1016.4195x; 2.2257x over previous
import jax
import jax.numpy as jnp
from jax.experimental import pallas as pl
from jax.experimental.pallas import tpu as pltpu

_OBS_DIM = 512
_HIDDEN_DIM = 512
_NUM_ENVS = 64
_N_AGENTS = 128
_DIST = 0.2
_EPP = 4


def _fused_env_kernel(pos_ref, post_ref, x_ref, w1_ref, b1_ref, w2_ref,
                      b2_ref, out_ref):
    masks = []
    degs = []
    aggs = []
    for i in range(_EPP):
        pos = pos_ref[i]
        post = post_ref[i]
        dx = pos[:, 0:1] - post[0:1, :]
        dy = pos[:, 1:2] - post[1:2, :]
        dist = jnp.sqrt(dx * dx + dy * dy)
        maskf = (dist <= _DIST).astype(jnp.float32)
        deg = jnp.maximum(jnp.sum(maskf, axis=1, keepdims=True), 1.0)
        masks.append(maskf)
        degs.append(deg)
        aggs.append(
            jnp.dot(maskf, x_ref[i], preferred_element_type=jnp.float32)
            / deg)

    agg = jnp.concatenate(aggs, axis=0)
    h1 = jnp.maximum(
        jnp.dot(agg, w1_ref[...], preferred_element_type=jnp.float32)
        + b1_ref[...], 0.0)

    aggs2 = []
    for i in range(_EPP):
        h1i = h1[i * _N_AGENTS:(i + 1) * _N_AGENTS, :]
        aggs2.append(
            jnp.dot(masks[i], h1i, preferred_element_type=jnp.float32)
            / degs[i])
    agg2 = jnp.concatenate(aggs2, axis=0)
    h2 = jnp.maximum(
        jnp.dot(agg2, w2_ref[...], preferred_element_type=jnp.float32)
        + b2_ref[...], 0.0)

    out_ref[...] = h2


def kernel(x, positions, W1, b1, W2, b2):
    num_envs, n_agents, feat = x.shape
    pos_t = positions.transpose(0, 2, 1)
    b1r = b1.reshape(1, _HIDDEN_DIM)
    b2r = b2.reshape(1, _HIDDEN_DIM)

    out = pl.pallas_call(
        _fused_env_kernel,
        grid=(num_envs // _EPP,),
        in_specs=[
            pl.BlockSpec((_EPP, n_agents, 2), lambda e: (e, 0, 0)),
            pl.BlockSpec((_EPP, 2, n_agents), lambda e: (e, 0, 0)),
            pl.BlockSpec((_EPP, n_agents, feat), lambda e: (e, 0, 0)),
            pl.BlockSpec((feat, _HIDDEN_DIM), lambda e: (0, 0)),
            pl.BlockSpec((1, _HIDDEN_DIM), lambda e: (0, 0)),
            pl.BlockSpec((_HIDDEN_DIM, _HIDDEN_DIM), lambda e: (0, 0)),
            pl.BlockSpec((1, _HIDDEN_DIM), lambda e: (0, 0)),
        ],
        out_specs=pl.BlockSpec((_EPP * n_agents, _HIDDEN_DIM),
                               lambda e: (e, 0)),
        out_shape=jax.ShapeDtypeStruct((num_envs * n_agents, _HIDDEN_DIM),
                                       jnp.float32),
        compiler_params=pltpu.CompilerParams(
            dimension_semantics=("parallel",)),
    )(positions, pos_t, x, W1, b1r, W2, b2r)
    return out


# speedup vs baseline: 1210.3011x; 1.1907x over previous
import jax
import jax.numpy as jnp
from jax.experimental import pallas as pl
from jax.experimental.pallas import tpu as pltpu

_OBS_DIM = 512
_HIDDEN_DIM = 512
_NUM_ENVS = 64
_N_AGENTS = 128
_DIST = 0.2
_EPP = 8


def _fused_env_kernel(pos_ref, post_ref, x_ref, w1_ref, b1_ref, w2_ref,
                      b2_ref, out_ref):
    masks = []
    degs = []
    aggs = []
    for i in range(_EPP):
        pos = pos_ref[i]
        post = post_ref[i]
        dx = pos[:, 0:1] - post[0:1, :]
        dy = pos[:, 1:2] - post[1:2, :]
        dist = jnp.sqrt(dx * dx + dy * dy)
        maskf = (dist <= _DIST).astype(jnp.float32)
        deg = jnp.maximum(jnp.sum(maskf, axis=1, keepdims=True), 1.0)
        masks.append(maskf)
        degs.append(deg)
        aggs.append(
            jnp.dot(maskf, x_ref[i], preferred_element_type=jnp.float32)
            / deg)

    agg = jnp.concatenate(aggs, axis=0)
    h1 = jnp.maximum(
        jnp.dot(agg, w1_ref[...], preferred_element_type=jnp.float32)
        + b1_ref[...], 0.0)

    aggs2 = []
    for i in range(_EPP):
        h1i = h1[i * _N_AGENTS:(i + 1) * _N_AGENTS, :]
        aggs2.append(
            jnp.dot(masks[i], h1i, preferred_element_type=jnp.float32)
            / degs[i])
    agg2 = jnp.concatenate(aggs2, axis=0)
    h2 = jnp.maximum(
        jnp.dot(agg2, w2_ref[...], preferred_element_type=jnp.float32)
        + b2_ref[...], 0.0)

    out_ref[...] = h2


def kernel(x, positions, W1, b1, W2, b2):
    num_envs, n_agents, feat = x.shape
    pos_t = positions.transpose(0, 2, 1)
    b1r = b1.reshape(1, _HIDDEN_DIM)
    b2r = b2.reshape(1, _HIDDEN_DIM)

    out = pl.pallas_call(
        _fused_env_kernel,
        grid=(num_envs // _EPP,),
        in_specs=[
            pl.BlockSpec((_EPP, n_agents, 2), lambda e: (e, 0, 0)),
            pl.BlockSpec((_EPP, 2, n_agents), lambda e: (e, 0, 0)),
            pl.BlockSpec((_EPP, n_agents, feat), lambda e: (e, 0, 0)),
            pl.BlockSpec((feat, _HIDDEN_DIM), lambda e: (0, 0)),
            pl.BlockSpec((1, _HIDDEN_DIM), lambda e: (0, 0)),
            pl.BlockSpec((_HIDDEN_DIM, _HIDDEN_DIM), lambda e: (0, 0)),
            pl.BlockSpec((1, _HIDDEN_DIM), lambda e: (0, 0)),
        ],
        out_specs=pl.BlockSpec((_EPP * n_agents, _HIDDEN_DIM),
                               lambda e: (e, 0)),
        out_shape=jax.ShapeDtypeStruct((num_envs * n_agents, _HIDDEN_DIM),
                                       jnp.float32),
        compiler_params=pltpu.CompilerParams(
            dimension_semantics=("parallel",)),
    )(positions, pos_t, x, W1, b1r, W2, b2r)
    return out


# speedup vs baseline: 1288.9469x; 1.0650x over previous
import jax
import jax.numpy as jnp
from jax.experimental import pallas as pl
from jax.experimental.pallas import tpu as pltpu

_OBS_DIM = 512
_HIDDEN_DIM = 512
_NUM_ENVS = 64
_N_AGENTS = 128
_DIST = 0.2
_EPP = 16


def _fused_env_kernel(pos_ref, post_ref, x_ref, w1_ref, b1_ref, w2_ref,
                      b2_ref, out_ref):
    masks = []
    degs = []
    aggs = []
    for i in range(_EPP):
        pos = pos_ref[i]
        post = post_ref[i]
        dx = pos[:, 0:1] - post[0:1, :]
        dy = pos[:, 1:2] - post[1:2, :]
        dist = jnp.sqrt(dx * dx + dy * dy)
        maskf = (dist <= _DIST).astype(jnp.float32)
        deg = jnp.maximum(jnp.sum(maskf, axis=1, keepdims=True), 1.0)
        masks.append(maskf)
        degs.append(deg)
        aggs.append(
            jnp.dot(maskf, x_ref[i], preferred_element_type=jnp.float32)
            / deg)

    agg = jnp.concatenate(aggs, axis=0)
    h1 = jnp.maximum(
        jnp.dot(agg, w1_ref[...], preferred_element_type=jnp.float32)
        + b1_ref[...], 0.0)

    aggs2 = []
    for i in range(_EPP):
        h1i = h1[i * _N_AGENTS:(i + 1) * _N_AGENTS, :]
        aggs2.append(
            jnp.dot(masks[i], h1i, preferred_element_type=jnp.float32)
            / degs[i])
    agg2 = jnp.concatenate(aggs2, axis=0)
    h2 = jnp.maximum(
        jnp.dot(agg2, w2_ref[...], preferred_element_type=jnp.float32)
        + b2_ref[...], 0.0)

    out_ref[...] = h2


def kernel(x, positions, W1, b1, W2, b2):
    num_envs, n_agents, feat = x.shape
    pos_t = positions.transpose(0, 2, 1)
    b1r = b1.reshape(1, _HIDDEN_DIM)
    b2r = b2.reshape(1, _HIDDEN_DIM)

    out = pl.pallas_call(
        _fused_env_kernel,
        grid=(num_envs // _EPP,),
        in_specs=[
            pl.BlockSpec((_EPP, n_agents, 2), lambda e: (e, 0, 0)),
            pl.BlockSpec((_EPP, 2, n_agents), lambda e: (e, 0, 0)),
            pl.BlockSpec((_EPP, n_agents, feat), lambda e: (e, 0, 0)),
            pl.BlockSpec((feat, _HIDDEN_DIM), lambda e: (0, 0)),
            pl.BlockSpec((1, _HIDDEN_DIM), lambda e: (0, 0)),
            pl.BlockSpec((_HIDDEN_DIM, _HIDDEN_DIM), lambda e: (0, 0)),
            pl.BlockSpec((1, _HIDDEN_DIM), lambda e: (0, 0)),
        ],
        out_specs=pl.BlockSpec((_EPP * n_agents, _HIDDEN_DIM),
                               lambda e: (e, 0)),
        out_shape=jax.ShapeDtypeStruct((num_envs * n_agents, _HIDDEN_DIM),
                                       jnp.float32),
        compiler_params=pltpu.CompilerParams(
            dimension_semantics=("parallel",)),
    )(positions, pos_t, x, W1, b1r, W2, b2r)
    return out
